# Initial kernel scaffold; baseline (speedup 1.0000x reference)
import jax
import jax.numpy as jnp
from jax.experimental import pallas as pl


def kernel(feats, edge_index, W, b, W_res, b_res, gamma, beta):
    raise NotImplementedError("write your pallas kernel here")


# speedup vs baseline: 4.4264x; 4.4264x over previous
import functools

import jax
import jax.numpy as jnp
from jax import lax
from jax.experimental import pallas as pl
from jax.experimental.pallas import tpu as pltpu
from jax.experimental.pallas import tpu_sc as plsc

_NC = 2
_NS = 16
_CH = 128
_EPS = 1e-5


def _sc_segment_sum(n_acc, d, n_chunks, rows_per_tile):
    mesh = plsc.VectorSubcoreMesh(core_axis_name="c", subcore_axis_name="s")

    @functools.partial(
        pl.kernel,
        mesh=mesh,
        out_type=jax.ShapeDtypeStruct((_NC, n_acc, d), jnp.float32),
        scratch_types=[
            pltpu.VMEM((n_chunks, _CH), jnp.int32),
            pltpu.VMEM((n_chunks, _CH), jnp.int32),
            pltpu.VMEM((_CH, d), jnp.float32),
            pltpu.VMEM_SHARED((n_acc, d), jnp.float32),
            pltpu.SemaphoreType.DMA,
        ],
    )
    def kern(feats_hbm, src_hbm, dst_hbm, zeros_hbm, out_hbm,
             src_v, dst_v, rows_v, acc, sem):
        cid = lax.axis_index("c")
        sid = lax.axis_index("s")
        wid = sid * _NC + cid
        r0 = sid * rows_per_tile
        pltpu.sync_copy(zeros_hbm, acc.at[pl.ds(r0, rows_per_tile)])
        pltpu.sync_copy(src_hbm.at[wid], src_v)
        pltpu.sync_copy(dst_hbm.at[wid], dst_v)
        plsc.subcore_barrier()

        @pl.loop(0, n_chunks)
        def _(j):
            pltpu.async_copy(feats_hbm.at[src_v.at[j]], rows_v, sem).wait()
            pltpu.sync_copy(rows_v, acc.at[dst_v.at[j]], add=True)

        plsc.subcore_barrier()
        pltpu.sync_copy(acc.at[pl.ds(r0, rows_per_tile)],
                        out_hbm.at[cid, pl.ds(r0, rows_per_tile)])

    return kern


def _tc_epilogue(p_ref, f_ref, w_ref, b_ref, wr_ref, br_ref, g_ref, bt_ref,
                 o_ref):
    n = f_ref.shape[0]
    agg = p_ref[0, :n, :] + p_ref[1, :n, :]
    h = jnp.dot(agg, w_ref[...], preferred_element_type=jnp.float32)
    h = jnp.maximum(h + b_ref[...], 0.0)
    res = jnp.dot(f_ref[...], wr_ref[...], preferred_element_type=jnp.float32)
    res = jnp.maximum(res + br_ref[...], 0.0)
    h = h + res
    mean = jnp.mean(h, axis=0, keepdims=True)
    c = h - mean
    var = jnp.mean(c * c, axis=0, keepdims=True)
    o_ref[...] = c * lax.rsqrt(var + _EPS) * g_ref[...] + bt_ref[...]


def kernel(feats, edge_index, W, b, W_res, b_res, gamma, beta):
    n, d = feats.shape
    e = edge_index.shape[1]
    nw = _NC * _NS

    rows_per_tile = -(-(n + 1) // (_NS * 8)) * 8
    n_acc = rows_per_tile * _NS
    e_per_tile = -(-e // (nw * _CH)) * _CH
    n_chunks = e_per_tile // _CH
    e_pad = e_per_tile * nw

    src = edge_index[0].astype(jnp.int32)
    dst = edge_index[1].astype(jnp.int32)
    if e_pad > e:
        fill = jnp.full((e_pad - e,), n, jnp.int32)
        src = jnp.concatenate([src, fill])
        dst = jnp.concatenate([dst, fill])
    src3 = src.reshape(nw, n_chunks, _CH)
    dst3 = dst.reshape(nw, n_chunks, _CH)
    feats_pad = jnp.zeros((n_acc, d), jnp.float32).at[:n].set(feats)
    zeros = jnp.zeros((rows_per_tile, d), jnp.float32)

    partials = _sc_segment_sum(n_acc, d, n_chunks, rows_per_tile)(
        feats_pad, src3, dst3, zeros)

    out = pl.pallas_call(
        _tc_epilogue,
        out_shape=jax.ShapeDtypeStruct((n, d), jnp.float32),
    )(partials, feats, W, b.reshape(1, d), W_res, b_res.reshape(1, d),
      gamma.reshape(1, d), beta.reshape(1, d))
    return out
